# Initial kernel scaffold; baseline (speedup 1.0000x reference)
import jax
import jax.numpy as jnp
from jax.experimental import pallas as pl


def kernel(x, edge_index, W, b):
    raise NotImplementedError("write your pallas kernel here")


# speedup vs baseline: 7.7322x; 7.7322x over previous
import functools

import jax
import jax.numpy as jnp
from jax import lax
from jax.experimental import pallas as pl
from jax.experimental.pallas import tpu as pltpu
from jax.experimental.pallas import tpu_sc as plsc

NC = 2
NS = 16
LANES = 128


def _mesh():
    return plsc.VectorSubcoreMesh(
        core_axis_name="c", subcore_axis_name="s",
        num_cores=NC, num_subcores=NS)


def _deg_kernel(np_pad, chunks, d):
    rps = np_pad // NS

    @functools.partial(
        pl.kernel,
        out_type=jax.ShapeDtypeStruct((NC, np_pad, d), jnp.float32),
        mesh=_mesh(),
        scratch_types=[
            pltpu.VMEM((chunks, LANES), jnp.int32),
            pltpu.VMEM((LANES,), jnp.int32),
            pltpu.VMEM((LANES, d), jnp.float32),
            pltpu.VMEM_SHARED((np_pad, d), jnp.float32),
        ],
    )
    def k(dst_hbm, zeros_hbm, out_hbm, dst_v, didx, ones_v, acc):
        c = lax.axis_index("c")
        s = lax.axis_index("s")
        w = s * NC + c
        pltpu.sync_copy(zeros_hbm.at[pl.ds(s * rps, rps)],
                        acc.at[pl.ds(s * rps, rps)])
        one16 = jnp.ones((16,), jnp.float32)

        def obody(i, carry):
            ones_v[i // (d // 16), pl.ds((i % (d // 16)) * 16, 16)] = one16
            return carry

        lax.fori_loop(0, LANES * (d // 16), obody, 0)
        pltpu.sync_copy(dst_hbm.at[w], dst_v)
        plsc.subcore_barrier()

        def body(j, carry):
            for kk in range(LANES // 16):
                didx[pl.ds(kk * 16, 16)] = dst_v[j, pl.ds(kk * 16, 16)]
            pltpu.sync_copy(ones_v, acc.at[didx], add=True)
            return carry

        lax.fori_loop(0, chunks, body, 0)
        plsc.subcore_barrier()
        pltpu.sync_copy(acc.at[pl.ds(s * rps, rps)],
                        out_hbm.at[c, pl.ds(s * rps, rps)])

    return k


def _gs_kernel(np_pad, chunks, d):
    rps = np_pad // NS
    halfch = chunks // 2
    pairs = halfch // 2

    @functools.partial(
        pl.kernel,
        out_type=jax.ShapeDtypeStruct((NC, np_pad, d), jnp.float32),
        mesh=_mesh(),
        scratch_types=[
            pltpu.VMEM((halfch, LANES), jnp.int32),
            pltpu.VMEM((halfch, LANES), jnp.int32),
            pltpu.VMEM((LANES,), jnp.int32),
            pltpu.VMEM((LANES, d), jnp.float32),
            pltpu.VMEM((LANES, d), jnp.float32),
            pltpu.VMEM_SHARED((np_pad, d), jnp.float32),
            pltpu.SemaphoreType.DMA,
            pltpu.SemaphoreType.DMA,
        ],
    )
    def k(g_hbm, src_hbm, dst_hbm, zeros_hbm, out_hbm,
          src_v, dst_v, didx, rows0, rows1, acc, sem0, sem1):
        c = lax.axis_index("c")
        s = lax.axis_index("s")
        w = s * NC + c
        pltpu.sync_copy(zeros_hbm.at[pl.ds(s * rps, rps)],
                        acc.at[pl.ds(s * rps, rps)])
        plsc.subcore_barrier()

        def stage_didx(j):
            for kk in range(LANES // 16):
                didx[pl.ds(kk * 16, 16)] = dst_v[j, pl.ds(kk * 16, 16)]

        for h in range(2):
            pltpu.sync_copy(src_hbm.at[w, pl.ds(h * halfch, halfch)], src_v)
            pltpu.sync_copy(dst_hbm.at[w, pl.ds(h * halfch, halfch)], dst_v)
            pltpu.async_copy(g_hbm.at[src_v.at[0]], rows0, sem0)

            def body(jj, carry):
                j = jj * 2
                pltpu.async_copy(g_hbm.at[src_v.at[j + 1]], rows1, sem1)
                stage_didx(j)
                pltpu.make_async_copy(g_hbm.at[src_v.at[j]], rows0, sem0).wait()
                pltpu.sync_copy(rows0, acc.at[didx], add=True)
                pltpu.async_copy(g_hbm.at[src_v.at[j + 2]], rows0, sem0)
                stage_didx(j + 1)
                pltpu.make_async_copy(
                    g_hbm.at[src_v.at[j + 1]], rows1, sem1).wait()
                pltpu.sync_copy(rows1, acc.at[didx], add=True)
                return carry

            lax.fori_loop(0, pairs - 1, body, 0)
            j = halfch - 2
            pltpu.async_copy(g_hbm.at[src_v.at[j + 1]], rows1, sem1)
            stage_didx(j)
            pltpu.make_async_copy(g_hbm.at[src_v.at[j]], rows0, sem0).wait()
            pltpu.sync_copy(rows0, acc.at[didx], add=True)
            stage_didx(j + 1)
            pltpu.make_async_copy(g_hbm.at[src_v.at[j + 1]], rows1, sem1).wait()
            pltpu.sync_copy(rows1, acc.at[didx], add=True)

        plsc.subcore_barrier()
        pltpu.sync_copy(acc.at[pl.ds(s * rps, rps)],
                        out_hbm.at[c, pl.ds(s * rps, rps)])

    return k


R_BLK = 2048


def _deg_block(p_ref):
    return p_ref[0, :, 0] + p_ref[1, :, 0] + 1.0


def _row_block(n):
    return R_BLK


def _tc1(n, np_pad, d):
    r = _row_block(n)

    def body(p_ref, x_ref, w_ref, h_ref, g_ref):
        deg = _deg_block(p_ref)
        dinv = lax.rsqrt(deg)
        h = jnp.dot(x_ref[...], w_ref[...], preferred_element_type=jnp.float32)
        h_ref[...] = h
        g_ref[...] = h * dinv[:, None]

    return pl.pallas_call(
        body,
        grid=(-(-n // r),),
        in_specs=[
            pl.BlockSpec((NC, r, d), lambda i: (0, i, 0)),
            pl.BlockSpec((r, d), lambda i: (i, 0)),
            pl.BlockSpec((d, d), lambda i: (0, 0)),
        ],
        out_specs=[
            pl.BlockSpec((r, d), lambda i: (i, 0)),
            pl.BlockSpec((r, d), lambda i: (i, 0)),
        ],
        out_shape=[
            jax.ShapeDtypeStruct((n, d), jnp.float32),
            jax.ShapeDtypeStruct((n, d), jnp.float32),
        ],
    )


def _tc2(n, np_pad, d):
    r = _row_block(n)

    def body(sp_ref, h_ref, p_ref, w_ref, b_ref, h2_ref, g2_ref):
        deg = _deg_block(p_ref)
        dinv = lax.rsqrt(deg)
        ssum = sp_ref[0] + sp_ref[1]
        z = (dinv[:, None] * ssum + (dinv * dinv)[:, None] * h_ref[...]
             + b_ref[...])
        z = jnp.maximum(z, 0.0)
        h2 = jnp.dot(z, w_ref[...], preferred_element_type=jnp.float32)
        h2_ref[...] = h2
        g2_ref[...] = h2 * dinv[:, None]

    return pl.pallas_call(
        body,
        grid=(-(-n // r),),
        in_specs=[
            pl.BlockSpec((NC, r, d), lambda i: (0, i, 0)),
            pl.BlockSpec((r, d), lambda i: (i, 0)),
            pl.BlockSpec((NC, r, d), lambda i: (0, i, 0)),
            pl.BlockSpec((d, d), lambda i: (0, 0)),
            pl.BlockSpec((1, d), lambda i: (0, 0)),
        ],
        out_specs=[
            pl.BlockSpec((r, d), lambda i: (i, 0)),
            pl.BlockSpec((r, d), lambda i: (i, 0)),
        ],
        out_shape=[
            jax.ShapeDtypeStruct((n, d), jnp.float32),
            jax.ShapeDtypeStruct((n, d), jnp.float32),
        ],
    )


def _tc3(n, np_pad, d):
    r = _row_block(n)

    def body(sp_ref, h2_ref, p_ref, b_ref, out_ref):
        deg = _deg_block(p_ref)
        dinv = lax.rsqrt(deg)
        ssum = sp_ref[0] + sp_ref[1]
        out_ref[...] = (dinv[:, None] * ssum
                        + (dinv * dinv)[:, None] * h2_ref[...] + b_ref[...])

    return pl.pallas_call(
        body,
        grid=(-(-n // r),),
        in_specs=[
            pl.BlockSpec((NC, r, d), lambda i: (0, i, 0)),
            pl.BlockSpec((r, d), lambda i: (i, 0)),
            pl.BlockSpec((NC, r, d), lambda i: (0, i, 0)),
            pl.BlockSpec((1, d), lambda i: (0, 0)),
        ],
        out_specs=pl.BlockSpec((r, d), lambda i: (i, 0)),
        out_shape=jax.ShapeDtypeStruct((n, d), jnp.float32),
    )


def kernel(x, edge_index, W, b):
    n, d = x.shape
    e = edge_index.shape[1]
    nw = NC * NS

    pair = 4 * nw * LANES
    ep = ((e + pair - 1) // pair) * pair
    pad = ep - e
    src_p = jnp.concatenate(
        [edge_index[0], jnp.zeros((pad,), edge_index.dtype)])
    dst_p = jnp.concatenate(
        [edge_index[1], jnp.full((pad,), n, edge_index.dtype)])
    chunks = ep // nw // LANES
    src3 = src_p.reshape(nw, chunks, LANES)
    dst3 = dst_p.reshape(nw, chunks, LANES)

    np_pad = ((n + 1 + R_BLK - 1) // R_BLK) * R_BLK
    zeros_d = jnp.zeros((np_pad, d), jnp.float32)
    b2 = b.reshape(1, d).astype(jnp.float32)

    deg_parts = _deg_kernel(np_pad, chunks, d)(dst3, zeros_d)
    h1, g1 = _tc1(n, np_pad, d)(deg_parts, x, W)
    s1 = _gs_kernel(np_pad, chunks, d)(g1, src3, dst3, zeros_d)
    h2, g2 = _tc2(n, np_pad, d)(s1, h1, deg_parts, W, b2)
    s2 = _gs_kernel(np_pad, chunks, d)(g2, src3, dst3, zeros_d)
    out = _tc3(n, np_pad, d)(s2, h2, deg_parts, b2)
    return out


# speedup vs baseline: 7.9396x; 1.0268x over previous
import functools

import jax
import jax.numpy as jnp
from jax import lax
from jax.experimental import pallas as pl
from jax.experimental.pallas import tpu as pltpu
from jax.experimental.pallas import tpu_sc as plsc

NC = 2
NS = 16
LANES = 128

CH_FAST, CH_SLOW = 136, 24
STAGES_FAST = (40, 32, 32, 32)
STAGES_SLOW = (24,)
CH_TOT = CH_FAST + CH_SLOW

R_BLK = 2048


def _mesh():
    return plsc.VectorSubcoreMesh(
        core_axis_name="c", subcore_axis_name="s",
        num_cores=NC, num_subcores=NS)


def _fill(buf, d, value):
    v16 = jnp.full((16,), value, jnp.float32)
    grp = d // 16

    def body(i, carry):
        buf[i // grp, pl.ds((i % grp) * 16, 16)] = v16
        return carry

    lax.fori_loop(0, LANES * grp, body, 0)


def _zero_acc(rows0, acc, d, s, rps):
    _fill(rows0, d, 0.0)
    for t in range(rps // LANES):
        pltpu.sync_copy(rows0, acc.at[pl.ds(s * rps + t * LANES, LANES)])


def _deg_kernel(np_pad, chunks, d):
    rps = np_pad // NS

    @functools.partial(
        pl.kernel,
        out_type=jax.ShapeDtypeStruct((NC, np_pad, d), jnp.float32),
        mesh=_mesh(),
        scratch_types=[
            pltpu.VMEM((chunks, LANES), jnp.int32),
            pltpu.VMEM((LANES,), jnp.int32),
            pltpu.VMEM((LANES, d), jnp.float32),
            pltpu.VMEM_SHARED((np_pad, d), jnp.float32),
        ],
    )
    def k(dst_hbm, out_hbm, dst_v, didx, ones_v, acc):
        c = lax.axis_index("c")
        s = lax.axis_index("s")
        w = s * NC + c
        _zero_acc(ones_v, acc, d, s, rps)
        _fill(ones_v, d, 1.0)
        pltpu.sync_copy(dst_hbm.at[w], dst_v)
        plsc.subcore_barrier()

        def body(j, carry):
            for kk in range(LANES // 16):
                didx[pl.ds(kk * 16, 16)] = dst_v[j, pl.ds(kk * 16, 16)]
            pltpu.sync_copy(ones_v, acc.at[didx], add=True)
            return carry

        lax.fori_loop(0, chunks, body, 0)
        plsc.subcore_barrier()
        pltpu.sync_copy(acc.at[pl.ds(s * rps, rps)],
                        out_hbm.at[c, pl.ds(s * rps, rps)])

    return k


def _gs_kernel(np_pad, d):
    rps = np_pad // NS
    stg_max = max(STAGES_FAST + STAGES_SLOW)

    @functools.partial(
        pl.kernel,
        out_type=jax.ShapeDtypeStruct((NC, np_pad, d), jnp.float32),
        mesh=_mesh(),
        scratch_types=[
            pltpu.VMEM((stg_max, LANES), jnp.int32),
            pltpu.VMEM((stg_max, LANES), jnp.int32),
            pltpu.VMEM((LANES,), jnp.int32),
            pltpu.VMEM((LANES, d), jnp.float32),
            pltpu.VMEM((LANES, d), jnp.float32),
            pltpu.VMEM_SHARED((np_pad, d), jnp.float32),
            pltpu.SemaphoreType.DMA,
            pltpu.SemaphoreType.DMA,
        ],
    )
    def k(g_hbm, src_hbm, dst_hbm, out_hbm,
          src_v, dst_v, didx, rows0, rows1, acc, sem0, sem1):
        c = lax.axis_index("c")
        s = lax.axis_index("s")
        _zero_acc(rows0, acc, d, s, rps)
        plsc.subcore_barrier()

        def stage_didx(j):
            for kk in range(LANES // 16):
                didx[pl.ds(kk * 16, 16)] = dst_v[j, pl.ds(kk * 16, 16)]

        def stage(off, stg):
            pairs = stg // 2
            pltpu.sync_copy(src_hbm.at[pl.ds(off, stg)],
                            src_v.at[pl.ds(0, stg)])
            pltpu.sync_copy(dst_hbm.at[pl.ds(off, stg)],
                            dst_v.at[pl.ds(0, stg)])
            pltpu.async_copy(g_hbm.at[src_v.at[0]], rows0, sem0)

            def body(jj, carry):
                j = jj * 2
                pltpu.async_copy(g_hbm.at[src_v.at[j + 1]], rows1, sem1)
                stage_didx(j)
                pltpu.make_async_copy(g_hbm.at[src_v.at[j]], rows0, sem0).wait()
                pltpu.sync_copy(rows0, acc.at[didx], add=True)
                pltpu.async_copy(g_hbm.at[src_v.at[j + 2]], rows0, sem0)
                stage_didx(j + 1)
                pltpu.make_async_copy(
                    g_hbm.at[src_v.at[j + 1]], rows1, sem1).wait()
                pltpu.sync_copy(rows1, acc.at[didx], add=True)
                return carry

            lax.fori_loop(0, pairs - 1, body, 0)
            j = stg - 2
            pltpu.async_copy(g_hbm.at[src_v.at[j + 1]], rows1, sem1)
            stage_didx(j)
            pltpu.make_async_copy(g_hbm.at[src_v.at[j]], rows0, sem0).wait()
            pltpu.sync_copy(rows0, acc.at[didx], add=True)
            stage_didx(j + 1)
            pltpu.make_async_copy(g_hbm.at[src_v.at[j + 1]], rows1, sem1).wait()
            pltpu.sync_copy(rows1, acc.at[didx], add=True)

        @pl.when(c == 0)
        def _():
            off = NS * CH_SLOW + s * CH_FAST
            for stg in STAGES_FAST:
                stage(off, stg)
                off += stg

        @pl.when(c == 1)
        def _():
            off = s * CH_SLOW
            for stg in STAGES_SLOW:
                stage(off, stg)
                off += stg

        plsc.subcore_barrier()
        pltpu.sync_copy(acc.at[pl.ds(s * rps, rps)],
                        out_hbm.at[c, pl.ds(s * rps, rps)])

    return k


def _deg_block(p_ref):
    return p_ref[0, :, 0] + p_ref[1, :, 0] + 1.0


def _tc1(n, np_pad, d):
    r = R_BLK

    def body(p_ref, x_ref, w_ref, h_ref, g_ref):
        deg = _deg_block(p_ref)
        dinv = lax.rsqrt(deg)
        h = jnp.dot(x_ref[...], w_ref[...], preferred_element_type=jnp.float32)
        h_ref[...] = h
        g_ref[...] = h * dinv[:, None]

    return pl.pallas_call(
        body,
        grid=(-(-n // r),),
        in_specs=[
            pl.BlockSpec((NC, r, d), lambda i: (0, i, 0)),
            pl.BlockSpec((r, d), lambda i: (i, 0)),
            pl.BlockSpec((d, d), lambda i: (0, 0)),
        ],
        out_specs=[
            pl.BlockSpec((r, d), lambda i: (i, 0)),
            pl.BlockSpec((r, d), lambda i: (i, 0)),
        ],
        out_shape=[
            jax.ShapeDtypeStruct((n, d), jnp.float32),
            jax.ShapeDtypeStruct((n, d), jnp.float32),
        ],
    )


def _tc2(n, np_pad, d):
    r = R_BLK

    def body(sp_ref, h_ref, p_ref, w_ref, b_ref, h2_ref, g2_ref):
        deg = _deg_block(p_ref)
        dinv = lax.rsqrt(deg)
        ssum = sp_ref[0] + sp_ref[1]
        z = (dinv[:, None] * ssum + (dinv * dinv)[:, None] * h_ref[...]
             + b_ref[...])
        z = jnp.maximum(z, 0.0)
        h2 = jnp.dot(z, w_ref[...], preferred_element_type=jnp.float32)
        h2_ref[...] = h2
        g2_ref[...] = h2 * dinv[:, None]

    return pl.pallas_call(
        body,
        grid=(-(-n // r),),
        in_specs=[
            pl.BlockSpec((NC, r, d), lambda i: (0, i, 0)),
            pl.BlockSpec((r, d), lambda i: (i, 0)),
            pl.BlockSpec((NC, r, d), lambda i: (0, i, 0)),
            pl.BlockSpec((d, d), lambda i: (0, 0)),
            pl.BlockSpec((1, d), lambda i: (0, 0)),
        ],
        out_specs=[
            pl.BlockSpec((r, d), lambda i: (i, 0)),
            pl.BlockSpec((r, d), lambda i: (i, 0)),
        ],
        out_shape=[
            jax.ShapeDtypeStruct((n, d), jnp.float32),
            jax.ShapeDtypeStruct((n, d), jnp.float32),
        ],
    )


def _tc3(n, np_pad, d):
    r = R_BLK

    def body(sp_ref, h2_ref, p_ref, b_ref, out_ref):
        deg = _deg_block(p_ref)
        dinv = lax.rsqrt(deg)
        ssum = sp_ref[0] + sp_ref[1]
        out_ref[...] = (dinv[:, None] * ssum
                        + (dinv * dinv)[:, None] * h2_ref[...] + b_ref[...])

    return pl.pallas_call(
        body,
        grid=(-(-n // r),),
        in_specs=[
            pl.BlockSpec((NC, r, d), lambda i: (0, i, 0)),
            pl.BlockSpec((r, d), lambda i: (i, 0)),
            pl.BlockSpec((NC, r, d), lambda i: (0, i, 0)),
            pl.BlockSpec((1, d), lambda i: (0, 0)),
        ],
        out_specs=pl.BlockSpec((r, d), lambda i: (i, 0)),
        out_shape=jax.ShapeDtypeStruct((n, d), jnp.float32),
    )


def kernel(x, edge_index, W, b):
    n, d = x.shape
    e = edge_index.shape[1]
    nw = NC * NS

    grain = NS * CH_TOT * LANES
    ep = ((e + grain - 1) // grain) * grain
    pad = ep - e
    src_p = jnp.concatenate(
        [edge_index[0], jnp.zeros((pad,), edge_index.dtype)])
    dst_p = jnp.concatenate(
        [edge_index[1], jnp.full((pad,), n, edge_index.dtype)])
    tot_chunks = ep // LANES
    src_f = src_p.reshape(tot_chunks, LANES)
    dst_f = dst_p.reshape(tot_chunks, LANES)
    chunks_deg = tot_chunks // nw
    dst3 = dst_p.reshape(nw, chunks_deg, LANES)

    np_pad = ((n + 1 + R_BLK - 1) // R_BLK) * R_BLK
    b2 = b.reshape(1, d).astype(jnp.float32)

    deg_parts = _deg_kernel(np_pad, chunks_deg, d)(dst3)
    h1, g1 = _tc1(n, np_pad, d)(deg_parts, x, W)
    s1 = _gs_kernel(np_pad, d)(g1, src_f, dst_f)
    h2, g2 = _tc2(n, np_pad, d)(s1, h1, deg_parts, W, b2)
    s2 = _gs_kernel(np_pad, d)(g2, src_f, dst_f)
    out = _tc3(n, np_pad, d)(s2, h2, deg_parts, b2)
    return out


# speedup vs baseline: 8.0793x; 1.0176x over previous
import functools

import jax
import jax.numpy as jnp
from jax import lax
from jax.experimental import pallas as pl
from jax.experimental.pallas import tpu as pltpu
from jax.experimental.pallas import tpu_sc as plsc

NC = 2
NS = 16
LANES = 128

CH_FAST, CH_SLOW = 136, 24
STAGES_FAST = (40, 32, 32, 32)
STAGES_SLOW = (24,)
CH_TOT = CH_FAST + CH_SLOW

R_BLK = 2048


def _mesh():
    return plsc.VectorSubcoreMesh(
        core_axis_name="c", subcore_axis_name="s",
        num_cores=NC, num_subcores=NS)


def _fill(buf, rows, d, value):
    v16 = jnp.full((16,), value, jnp.float32)
    grp = d // 16

    def body(i, carry):
        buf[i // grp, pl.ds((i % grp) * 16, 16)] = v16
        return carry

    lax.fori_loop(0, rows * grp, body, 0)


def _zero_acc(buf, rows, acc, d, s, rps):
    _fill(buf, rows, d, 0.0)
    for t in range(rps // rows):
        pltpu.sync_copy(buf, acc.at[pl.ds(s * rps + t * rows, rows)])


def _deg_kernel(np_pad, chunks, d):
    rps = np_pad // NS

    @functools.partial(
        pl.kernel,
        out_type=jax.ShapeDtypeStruct((NC, np_pad, d), jnp.float32),
        mesh=_mesh(),
        scratch_types=[
            pltpu.VMEM((chunks, LANES), jnp.int32),
            pltpu.VMEM((LANES,), jnp.int32),
            pltpu.VMEM((LANES, d), jnp.float32),
            pltpu.VMEM_SHARED((np_pad, d), jnp.float32),
        ],
    )
    def k(dst_hbm, out_hbm, dst_v, didx, ones_v, acc):
        c = lax.axis_index("c")
        s = lax.axis_index("s")
        w = s * NC + c
        _zero_acc(ones_v, LANES, acc, d, s, rps)
        _fill(ones_v, LANES, d, 1.0)
        pltpu.sync_copy(dst_hbm.at[w], dst_v)
        plsc.subcore_barrier()

        def body(j, carry):
            for kk in range(LANES // 16):
                didx[pl.ds(kk * 16, 16)] = dst_v[j, pl.ds(kk * 16, 16)]
            pltpu.sync_copy(ones_v, acc.at[didx], add=True)
            return carry

        lax.fori_loop(0, chunks, body, 0)
        plsc.subcore_barrier()
        pltpu.sync_copy(acc.at[pl.ds(s * rps, rps)],
                        out_hbm.at[c, pl.ds(s * rps, rps)])

    return k


def _gs_kernel(np_pad, d):
    rps = np_pad // NS
    stg_max = max(STAGES_FAST + STAGES_SLOW)

    @functools.partial(
        pl.kernel,
        out_type=jax.ShapeDtypeStruct((NC, np_pad, d), jnp.float32),
        mesh=_mesh(),
        scratch_types=[
            pltpu.VMEM((stg_max, LANES), jnp.int32),
            pltpu.VMEM((stg_max, LANES), jnp.int32),
            pltpu.VMEM((64,), jnp.int32),
            pltpu.VMEM((64, d), jnp.float32),
            pltpu.VMEM((64, d), jnp.float32),
            pltpu.VMEM((64, d), jnp.float32),
            pltpu.VMEM((64, d), jnp.float32),
            pltpu.VMEM_SHARED((np_pad, d), jnp.float32),
            pltpu.SemaphoreType.DMA,
            pltpu.SemaphoreType.DMA,
            pltpu.SemaphoreType.DMA,
            pltpu.SemaphoreType.DMA,
        ],
    )
    def k(g_hbm, src_hbm, dst_hbm, out_hbm,
          src_v, dst_v, didx, r0, r1, r2, r3, acc, s0, s1, s2, s3):
        c = lax.axis_index("c")
        s = lax.axis_index("s")
        bufs = (r0, r1, r2, r3)
        sems = (s0, s1, s2, s3)
        _zero_acc(r0, 64, acc, d, s, rps)
        plsc.subcore_barrier()

        def sidx(j, b):
            return src_v.at[j, pl.ds((b % 2) * 64, 64)]

        def stage_didx(j, b):
            for kk in range(4):
                didx[pl.ds(kk * 16, 16)] = (
                    dst_v[j, pl.ds((b % 2) * 64 + kk * 16, 16)])

        def stage(off, stg):
            nsub = 2 * stg
            groups = nsub // 4
            pltpu.sync_copy(src_hbm.at[pl.ds(off, stg)],
                            src_v.at[pl.ds(0, stg)])
            pltpu.sync_copy(dst_hbm.at[pl.ds(off, stg)],
                            dst_v.at[pl.ds(0, stg)])
            for b in range(4):
                pltpu.async_copy(g_hbm.at[sidx(b // 2, b)], bufs[b], sems[b])

            def body(gg, carry):
                for b in range(4):
                    q = gg * 4 + b
                    j = gg * 2 + b // 2
                    pltpu.make_async_copy(
                        g_hbm.at[sidx(j, b)], bufs[b], sems[b]).wait()
                    stage_didx(j, b)
                    pltpu.sync_copy(bufs[b], acc.at[didx], add=True)
                    jn = j + 2
                    pltpu.async_copy(g_hbm.at[sidx(jn, b)], bufs[b], sems[b])
                return carry

            lax.fori_loop(0, groups - 1, body, 0)
            for b in range(4):
                j = (groups - 1) * 2 + b // 2
                pltpu.make_async_copy(
                    g_hbm.at[sidx(j, b)], bufs[b], sems[b]).wait()
                stage_didx(j, b)
                pltpu.sync_copy(bufs[b], acc.at[didx], add=True)

        @pl.when(c == 0)
        def _():
            off = NS * CH_SLOW + s * CH_FAST
            for stg in STAGES_FAST:
                stage(off, stg)
                off += stg

        @pl.when(c == 1)
        def _():
            off = s * CH_SLOW
            for stg in STAGES_SLOW:
                stage(off, stg)
                off += stg

        plsc.subcore_barrier()
        pltpu.sync_copy(acc.at[pl.ds(s * rps, rps)],
                        out_hbm.at[c, pl.ds(s * rps, rps)])

    return k


def _deg_block(p_ref):
    return p_ref[0, :, 0] + p_ref[1, :, 0] + 1.0


def _tc1(n, np_pad, d):
    r = R_BLK

    def body(p_ref, x_ref, w_ref, h_ref, g_ref):
        deg = _deg_block(p_ref)
        dinv = lax.rsqrt(deg)
        h = jnp.dot(x_ref[...], w_ref[...], preferred_element_type=jnp.float32)
        h_ref[...] = h
        g_ref[...] = h * dinv[:, None]

    return pl.pallas_call(
        body,
        grid=(-(-n // r),),
        in_specs=[
            pl.BlockSpec((NC, r, d), lambda i: (0, i, 0)),
            pl.BlockSpec((r, d), lambda i: (i, 0)),
            pl.BlockSpec((d, d), lambda i: (0, 0)),
        ],
        out_specs=[
            pl.BlockSpec((r, d), lambda i: (i, 0)),
            pl.BlockSpec((r, d), lambda i: (i, 0)),
        ],
        out_shape=[
            jax.ShapeDtypeStruct((n, d), jnp.float32),
            jax.ShapeDtypeStruct((n, d), jnp.float32),
        ],
    )


def _tc2(n, np_pad, d):
    r = R_BLK

    def body(sp_ref, h_ref, p_ref, w_ref, b_ref, h2_ref, g2_ref):
        deg = _deg_block(p_ref)
        dinv = lax.rsqrt(deg)
        ssum = sp_ref[0] + sp_ref[1]
        z = (dinv[:, None] * ssum + (dinv * dinv)[:, None] * h_ref[...]
             + b_ref[...])
        z = jnp.maximum(z, 0.0)
        h2 = jnp.dot(z, w_ref[...], preferred_element_type=jnp.float32)
        h2_ref[...] = h2
        g2_ref[...] = h2 * dinv[:, None]

    return pl.pallas_call(
        body,
        grid=(-(-n // r),),
        in_specs=[
            pl.BlockSpec((NC, r, d), lambda i: (0, i, 0)),
            pl.BlockSpec((r, d), lambda i: (i, 0)),
            pl.BlockSpec((NC, r, d), lambda i: (0, i, 0)),
            pl.BlockSpec((d, d), lambda i: (0, 0)),
            pl.BlockSpec((1, d), lambda i: (0, 0)),
        ],
        out_specs=[
            pl.BlockSpec((r, d), lambda i: (i, 0)),
            pl.BlockSpec((r, d), lambda i: (i, 0)),
        ],
        out_shape=[
            jax.ShapeDtypeStruct((n, d), jnp.float32),
            jax.ShapeDtypeStruct((n, d), jnp.float32),
        ],
    )


def _tc3(n, np_pad, d):
    r = R_BLK

    def body(sp_ref, h2_ref, p_ref, b_ref, out_ref):
        deg = _deg_block(p_ref)
        dinv = lax.rsqrt(deg)
        ssum = sp_ref[0] + sp_ref[1]
        out_ref[...] = (dinv[:, None] * ssum
                        + (dinv * dinv)[:, None] * h2_ref[...] + b_ref[...])

    return pl.pallas_call(
        body,
        grid=(-(-n // r),),
        in_specs=[
            pl.BlockSpec((NC, r, d), lambda i: (0, i, 0)),
            pl.BlockSpec((r, d), lambda i: (i, 0)),
            pl.BlockSpec((NC, r, d), lambda i: (0, i, 0)),
            pl.BlockSpec((1, d), lambda i: (0, 0)),
        ],
        out_specs=pl.BlockSpec((r, d), lambda i: (i, 0)),
        out_shape=jax.ShapeDtypeStruct((n, d), jnp.float32),
    )


def kernel(x, edge_index, W, b):
    n, d = x.shape
    e = edge_index.shape[1]
    nw = NC * NS

    grain = NS * CH_TOT * LANES
    ep = ((e + grain - 1) // grain) * grain
    pad = ep - e
    src_p = jnp.concatenate(
        [edge_index[0], jnp.zeros((pad,), edge_index.dtype)])
    dst_p = jnp.concatenate(
        [edge_index[1], jnp.full((pad,), n, edge_index.dtype)])
    tot_chunks = ep // LANES
    src_f = src_p.reshape(tot_chunks, LANES)
    dst_f = dst_p.reshape(tot_chunks, LANES)
    chunks_deg = tot_chunks // nw
    dst3 = dst_p.reshape(nw, chunks_deg, LANES)

    np_pad = ((n + 1 + R_BLK - 1) // R_BLK) * R_BLK
    b2 = b.reshape(1, d).astype(jnp.float32)

    deg_parts = _deg_kernel(np_pad, chunks_deg, d)(dst3)
    h1, g1 = _tc1(n, np_pad, d)(deg_parts, x, W)
    s1 = _gs_kernel(np_pad, d)(g1, src_f, dst_f)
    h2, g2 = _tc2(n, np_pad, d)(s1, h1, deg_parts, W, b2)
    s2 = _gs_kernel(np_pad, d)(g2, src_f, dst_f)
    out = _tc3(n, np_pad, d)(s2, h2, deg_parts, b2)
    return out
